# Initial kernel scaffold; baseline (speedup 1.0000x reference)
import jax
import jax.numpy as jnp
from jax.experimental import pallas as pl


def kernel(token_ids, edge_src, emb, W_ih, b_ih, W_oh, b_oh, W_uh, b_uh, W_fh, b_fh, ln_g, ln_b, W_fc, b_fc):
    raise NotImplementedError("write your pallas kernel here")


# speedup vs baseline: 1.9046x; 1.9046x over previous
import functools

import jax
import jax.numpy as jnp
from jax import lax
from jax.experimental import pallas as pl
from jax.experimental.pallas import tpu as pltpu
from jax.experimental.pallas import tpu_sc as plsc

N, K, D, V, C = 10000, 32, 128, 100000, 104

NC, NS = 2, 16
NW = NC * NS
NPT = 320
N2 = NW * NPT
CH = 4
NCH = NPT // CH
E2 = N2 * K
NV = D // 16


def _sc_gather_sum(token_ids, edge_src_pad, emb):
    mesh = plsc.VectorSubcoreMesh(
        core_axis_name="c", subcore_axis_name="s", num_cores=NC, num_subcores=NS
    )

    @functools.partial(
        pl.kernel,
        out_type=jax.ShapeDtypeStruct((N2, D), jnp.float32),
        mesh=mesh,
        compiler_params=pltpu.CompilerParams(needs_layout_passes=False),
        scratch_types=[
            pltpu.VMEM((N,), jnp.int32),
            pltpu.VMEM((NPT * K,), jnp.int32),
            pltpu.VMEM((NCH, CH * K), jnp.int32),
            pltpu.VMEM((CH * K, D), jnp.float32),
            pltpu.VMEM((NPT, D), jnp.float32),
            pltpu.SemaphoreType.DMA,
        ],
    )
    def body(tok_hbm, edge_hbm, emb_hbm, out_hbm,
             tok_v, es_v, idx_v, mail_v, acc_v, sem):
        wid = lax.axis_index("s") * NC + lax.axis_index("c")

        pltpu.sync_copy(tok_hbm, tok_v)
        pltpu.sync_copy(edge_hbm.at[pl.ds(wid * (NPT * K), NPT * K)], es_v)

        def compose(b, carry):
            for q in range(CH * K // 16):
                e = es_v[pl.ds(b * (CH * K) + q * 16, 16)]
                idx_v[b, pl.ds(q * 16, 16)] = plsc.load_gather(tok_v, [e])
            return carry
        lax.fori_loop(0, NCH, compose, 0)

        def chunk(b, carry):
            pltpu.async_copy(emb_hbm.at[idx_v.at[b]], mail_v, sem).wait()
            for c in range(CH):
                base = c * K
                acc = tuple(mail_v[base, pl.ds(j * 16, 16)]
                            for j in range(NV))

                def red(k, a):
                    return tuple(a[j] + mail_v[base + k, pl.ds(j * 16, 16)]
                                 for j in range(NV))
                acc = lax.fori_loop(1, K - 1, red, acc)
                for j in range(NV):
                    acc_v[b * CH + c, pl.ds(j * 16, 16)] = acc[j]
            return carry

        lax.fori_loop(0, NCH, chunk, 0)
        pltpu.sync_copy(acc_v, out_hbm.at[pl.ds(wid * NPT, NPT)])

    return body(token_ids, edge_src_pad, emb)


def _tc_dense(s, w_all, b_all, ln_g2, ln_b2, w_fc, b_fc2):
    BN = 512

    def body(s_ref, wall_ref, ball_ref, lng_ref, lnb_ref, wfc_ref, bfc_ref,
             out_ref):
        x = s_ref[...]
        g = jnp.dot(x, wall_ref[...], preferred_element_type=jnp.float32)
        g = g + ball_ref[...]
        i = jax.nn.sigmoid(g[:, :D])
        o = jax.nn.sigmoid(g[:, D:2 * D])
        u = jnp.tanh(g[:, 2 * D:])
        h = o * jnp.tanh(i * u)
        mu = jnp.mean(h, axis=-1, keepdims=True)
        var = jnp.mean(jnp.square(h - mu), axis=-1, keepdims=True)
        hn = (h - mu) / jnp.sqrt(var + 1e-5) * lng_ref[...] + lnb_ref[...]
        out_ref[...] = (
            jnp.dot(hn, wfc_ref[...], preferred_element_type=jnp.float32)
            + bfc_ref[...])

    return pl.pallas_call(
        body,
        grid=(N2 // BN,),
        in_specs=[
            pl.BlockSpec((BN, D), lambda i: (i, 0)),
            pl.BlockSpec((D, 3 * D), lambda i: (0, 0)),
            pl.BlockSpec((1, 3 * D), lambda i: (0, 0)),
            pl.BlockSpec((1, D), lambda i: (0, 0)),
            pl.BlockSpec((1, D), lambda i: (0, 0)),
            pl.BlockSpec((D, 128), lambda i: (0, 0)),
            pl.BlockSpec((1, 128), lambda i: (0, 0)),
        ],
        out_specs=pl.BlockSpec((BN, 128), lambda i: (i, 0)),
        out_shape=jax.ShapeDtypeStruct((N2, 128), jnp.float32),
    )(s, w_all, b_all, ln_g2, ln_b2, w_fc, b_fc2)


def kernel(token_ids, edge_src, emb, W_ih, b_ih, W_oh, b_oh, W_uh, b_uh,
           W_fh, b_fh, ln_g, ln_b, W_fc, b_fc):
    token_ids = token_ids.astype(jnp.int32)
    edge_src = edge_src.astype(jnp.int32)
    edge_pad = jnp.pad(edge_src, (0, E2 - N * K))

    s = _sc_gather_sum(token_ids, edge_pad, emb)

    w_all = jnp.concatenate([W_ih.T, W_oh.T, W_uh.T], axis=1)
    b_all = jnp.concatenate([b_ih, b_oh, b_uh])[None, :]
    w_fc_p = jnp.zeros((D, 128), jnp.float32).at[:, :C].set(W_fc.T)
    b_fc_p = jnp.zeros((1, 128), jnp.float32).at[0, :C].set(b_fc)

    out = _tc_dense(s, w_all, b_all, ln_g[None, :], ln_b[None, :],
                    w_fc_p, b_fc_p)
    return out[:N, :C]


# speedup vs baseline: 2.0443x; 1.0734x over previous
import functools

import jax
import jax.numpy as jnp
from jax import lax
from jax.experimental import pallas as pl
from jax.experimental.pallas import tpu as pltpu
from jax.experimental.pallas import tpu_sc as plsc

N, K, D, V, C = 10000, 32, 128, 100000, 104

NC, NS = 2, 16
NW = NC * NS
NPT = 320
N2 = NW * NPT
CH = 2
NCH = NPT // CH
E2 = N2 * K
NV = D // 16


def _sc_gather_sum(token_ids, edge_src_pad, emb):
    mesh = plsc.VectorSubcoreMesh(
        core_axis_name="c", subcore_axis_name="s", num_cores=NC, num_subcores=NS
    )
    QG = 4
    RG = CH * K
    G = NCH // QG
    assert G % 2 == 0

    @functools.partial(
        pl.kernel,
        out_type=jax.ShapeDtypeStruct((N2, D), jnp.float32),
        mesh=mesh,
        compiler_params=pltpu.CompilerParams(needs_layout_passes=False),
        scratch_types=[
            pltpu.VMEM((N,), jnp.int32),
            pltpu.VMEM((NPT * K,), jnp.int32),
            pltpu.VMEM((2, QG, RG, D), jnp.float32),
            pltpu.VMEM((2, QG * CH, D), jnp.float32),
            pltpu.SemaphoreType.DMA,
            pltpu.SemaphoreType.DMA,
            pltpu.SemaphoreType.DMA,
            pltpu.SemaphoreType.DMA,
        ],
    )
    def body(tok_hbm, edge_hbm, emb_hbm, out_hbm,
             tok_v, eidx_v, mail_v, acc_v, sem0, sem1, semo0, semo1):
        wid = lax.axis_index("s") * NC + lax.axis_index("c")
        sems = (sem0, sem1)
        semos = (semo0, semo1)
        GR = QG * CH

        pltpu.sync_copy(tok_hbm, tok_v)
        pltpu.sync_copy(edge_hbm.at[pl.ds(wid * (NPT * K), NPT * K)], eidx_v)

        def compose(t, carry):
            for q in range(8):
                e = eidx_v[pl.ds(t * 128 + q * 16, 16)]
                eidx_v[pl.ds(t * 128 + q * 16, 16)] = plsc.load_gather(
                    tok_v, [e])
            return carry
        lax.fori_loop(0, NPT * K // 128, compose, 0)

        def fire(g, slot):
            for q in range(QG):
                pltpu.async_copy(
                    emb_hbm.at[eidx_v.at[pl.ds((g * QG + q) * RG, RG)]],
                    mail_v.at[slot, q], sems[slot])

        def drain(g, slot):
            for q in range(QG):
                pltpu.make_async_copy(
                    emb_hbm.at[eidx_v.at[pl.ds((g * QG + q) * RG, RG)]],
                    mail_v.at[slot, q], sems[slot]).wait()

        def reduce_group(slot):
            for q in range(QG):
                for c in range(CH):
                    acc = tuple(mail_v[slot, q, c * K, pl.ds(j * 16, 16)]
                                for j in range(NV))

                    def red(k, a, q=q, c=c):
                        return tuple(
                            a[j] + mail_v[slot, q, c * K + k,
                                          pl.ds(j * 16, 16)]
                            for j in range(NV))
                    acc = lax.fori_loop(1, K - 1, red, acc, unroll=5)
                    for j in range(NV):
                        acc_v[slot, q * CH + c, pl.ds(j * 16, 16)] = acc[j]

        def fire_out(g, slot):
            pltpu.async_copy(
                acc_v.at[slot],
                out_hbm.at[pl.ds(wid * NPT + g * GR, GR)], semos[slot])

        def drain_out(g, slot):
            pltpu.make_async_copy(
                acc_v.at[slot],
                out_hbm.at[pl.ds(wid * NPT + g * GR, GR)], semos[slot]).wait()

        fire(0, 0)

        def pair(p, carry):
            g0 = 2 * p
            fire(g0 + 1, 1)
            drain(g0, 0)

            @pl.when(p > 0)
            def _():
                drain_out(g0 - 2, 0)
            reduce_group(0)
            fire_out(g0, 0)

            @pl.when(g0 + 2 < G)
            def _():
                fire(g0 + 2, 0)
            drain(g0 + 1, 1)

            @pl.when(p > 0)
            def _():
                drain_out(g0 - 1, 1)
            reduce_group(1)
            fire_out(g0 + 1, 1)
            return carry

        lax.fori_loop(0, G // 2, pair, 0)
        drain_out(G - 2, 0)
        drain_out(G - 1, 1)

    return body(token_ids, edge_src_pad, emb)


def _tc_dense(s, w_all, b_all, ln_g2, ln_b2, w_fc, b_fc2):
    BN = 512

    def body(s_ref, wall_ref, ball_ref, lng_ref, lnb_ref, wfc_ref, bfc_ref,
             out_ref):
        x = s_ref[...]
        g = jnp.dot(x, wall_ref[...], preferred_element_type=jnp.float32)
        g = g + ball_ref[...]
        i = jax.nn.sigmoid(g[:, :D])
        o = jax.nn.sigmoid(g[:, D:2 * D])
        u = jnp.tanh(g[:, 2 * D:])
        h = o * jnp.tanh(i * u)
        mu = jnp.mean(h, axis=-1, keepdims=True)
        var = jnp.mean(jnp.square(h - mu), axis=-1, keepdims=True)
        hn = (h - mu) / jnp.sqrt(var + 1e-5) * lng_ref[...] + lnb_ref[...]
        out_ref[...] = (
            jnp.dot(hn, wfc_ref[...], preferred_element_type=jnp.float32)
            + bfc_ref[...])

    return pl.pallas_call(
        body,
        grid=(N2 // BN,),
        in_specs=[
            pl.BlockSpec((BN, D), lambda i: (i, 0)),
            pl.BlockSpec((D, 3 * D), lambda i: (0, 0)),
            pl.BlockSpec((1, 3 * D), lambda i: (0, 0)),
            pl.BlockSpec((1, D), lambda i: (0, 0)),
            pl.BlockSpec((1, D), lambda i: (0, 0)),
            pl.BlockSpec((D, 128), lambda i: (0, 0)),
            pl.BlockSpec((1, 128), lambda i: (0, 0)),
        ],
        out_specs=pl.BlockSpec((BN, 128), lambda i: (i, 0)),
        out_shape=jax.ShapeDtypeStruct((N2, 128), jnp.float32),
    )(s, w_all, b_all, ln_g2, ln_b2, w_fc, b_fc2)


def kernel(token_ids, edge_src, emb, W_ih, b_ih, W_oh, b_oh, W_uh, b_uh,
           W_fh, b_fh, ln_g, ln_b, W_fc, b_fc):
    token_ids = token_ids.astype(jnp.int32)
    edge_src = edge_src.astype(jnp.int32)
    edge_pad = jnp.pad(edge_src, (0, E2 - N * K))

    s = _sc_gather_sum(token_ids, edge_pad, emb)

    w_all = jnp.concatenate([W_ih.T, W_oh.T, W_uh.T], axis=1)
    b_all = jnp.concatenate([b_ih, b_oh, b_uh])[None, :]
    w_fc_p = jnp.zeros((D, 128), jnp.float32).at[:, :C].set(W_fc.T)
    b_fc_p = jnp.zeros((1, 128), jnp.float32).at[0, :C].set(b_fc)

    out = _tc_dense(s, w_all, b_all, ln_g[None, :], ln_b[None, :],
                    w_fc_p, b_fc_p)
    return out[:N, :C]
